# baseline (speedup 1.0000x reference)
import functools

import jax
import jax.numpy as jnp
from jax import lax
from jax.experimental import pallas as pl
from jax.experimental.pallas import tpu as pltpu
from jax.experimental.pallas import tpu_sc as plsc


B = 16384
L = 200
DU = 32
DF = 64
DC = 32
NG = 4
CG = 16
NW = 32
CHUNK = 32
SCHUNK = 128
NJ = L // 16
REM = L - NJ * 16
PCG = 8
PSTRIDE = 9
ZROW = 4000
NSPLIT = 4
NH = B // NSPLIT
SROWS = B // NW
NSC = SROWS // SCHUNK


def _lookup_body(user_r, feed_r, city_r, utab_r, ftab_r, ctab_r,
                 uo_r, fo_r, co_r,
                 uidx, fidx, cidx, ubuf, fbuf, cbuf, sg):
    wid = lax.axis_index("s") * 2 + lax.axis_index("c")
    r0 = wid * SROWS
    pltpu.sync_copy(user_r.at[pl.ds(r0, SROWS)], uidx)
    pltpu.sync_copy(feed_r.at[pl.ds(r0, SROWS)], fidx)
    pltpu.sync_copy(city_r.at[pl.ds(r0, SROWS)], cidx)

    def lookup_dmas():
        for sc in range(NSC):
            lo = sc * SCHUNK
            yield pltpu.make_async_copy(
                utab_r.at[uidx.at[pl.ds(lo, SCHUNK)]],
                ubuf.at[pl.ds(lo, SCHUNK), :], sg)
            yield pltpu.make_async_copy(
                ftab_r.at[fidx.at[pl.ds(lo, SCHUNK)]],
                fbuf.at[pl.ds(lo, SCHUNK), :], sg)
            yield pltpu.make_async_copy(
                ctab_r.at[cidx.at[pl.ds(lo, SCHUNK)]],
                cbuf.at[pl.ds(lo, SCHUNK), :], sg)

    for d in lookup_dmas():
        d.start()
    for d in lookup_dmas():
        d.wait()
    pltpu.sync_copy(ubuf, uo_r.at[pl.ds(r0, SROWS), :])
    pltpu.sync_copy(fbuf, fo_r.at[pl.ds(r0, SROWS), :])
    pltpu.sync_copy(cbuf, co_r.at[pl.ds(r0, SROWS), :])


@functools.lru_cache(maxsize=None)
def _lookup_call():
    mesh = plsc.VectorSubcoreMesh(core_axis_name="c", subcore_axis_name="s")
    return pl.kernel(
        _lookup_body,
        out_type=(
            jax.ShapeDtypeStruct((B, DU), jnp.float32),
            jax.ShapeDtypeStruct((B, DF), jnp.float32),
            jax.ShapeDtypeStruct((B, DC), jnp.float32),
        ),
        mesh=mesh,
        compiler_params=pltpu.CompilerParams(
            needs_layout_passes=False, use_tc_tiling_on_sc=False),
        scratch_types=[
            pltpu.VMEM((SROWS,), jnp.int32),
            pltpu.VMEM((SROWS,), jnp.int32),
            pltpu.VMEM((SROWS,), jnp.int32),
            pltpu.VMEM((SROWS, DU), jnp.float32),
            pltpu.VMEM((SROWS, DF), jnp.float32),
            pltpu.VMEM((SROWS, DC), jnp.float32),
            pltpu.SemaphoreType.DMA,
        ],
    )


def _make_pool_body(half):
    pool_rows = NH // (NW // NG)
    niter = pool_rows // CHUNK // 2

    def _pool_body(item_r, fg_r, po_r,
                   tbl, ib0, ib1, tbuf, ob0, ob1, si0, si1, so0, so1):
        wid = lax.axis_index("s") * 2 + lax.axis_index("c")
        g = lax.rem(wid, NG)
        q = lax.div(wid, NG)

        pltpu.sync_copy(fg_r.at[g], tbl)

        lane = lax.iota(jnp.int32, 16)
        lane_keep = lane >= (16 - REM)
        lane16 = lane * 16

        rows_abs = half * NH + q * pool_rows
        rows_loc = q * pool_rows

        def idx_dma(ch, ib, sem):
            src = item_r.at[pl.ds(rows_abs + ch * CHUNK, CHUNK), :]
            return pltpu.make_async_copy(src, ib, sem)

        def out_dma(ch, ob, sem):
            dst = po_r.at[g, pl.ds(rows_loc + ch * CHUNK, CHUNK), :]
            return pltpu.make_async_copy(ob, dst, sem)

        def process_chunk(ib, ob):
            def row_body(rl, carry):
                accs = [jnp.zeros((32,), jnp.bfloat16) for _ in range(PCG)]
                for j in range(NJ + 1):
                    iv = ib[rl, pl.ds(j * 16 if j < NJ else L - 16, 16)]
                    if j == NJ:
                        iv = jnp.where(lane_keep, iv, ZROW)
                    bidx = iv * PSTRIDE
                    for cc in range(PCG):
                        w = plsc.load_gather(tbl, [bidx + cc])
                        accs[cc] = accs[cc] + plsc.bitcast(w, jnp.bfloat16)
                for cc in range(PCG):
                    a, b = plsc.unpack(
                        accs[cc], format=plsc.PackFormat.INTERLEAVED)
                    tbuf[pl.ds((2 * cc) * 16, 16)] = a
                    tbuf[pl.ds((2 * cc + 1) * 16, 16)] = b
                pv = jnp.zeros((16,), jnp.float32)
                for j in range(16):
                    pv = pv + plsc.load_gather(
                        tbuf, [lane16 + ((lane + j) & 15)])
                ob[rl, :] = pv
                return carry
            lax.fori_loop(0, CHUNK, row_body, 0)

        idx_dma(0, ib0, si0).start()

        def iter_body(it, carry):
            idx_dma(0, ib0, si0).wait()
            idx_dma(2 * it + 1, ib1, si1).start()

            @pl.when(it > 0)
            def _():
                out_dma(0, ob0, so0).wait()

            process_chunk(ib0, ob0)
            out_dma(2 * it, ob0, so0).start()

            idx_dma(0, ib1, si1).wait()

            @pl.when(it < niter - 1)
            def _():
                idx_dma(2 * it + 2, ib0, si0).start()

            @pl.when(it > 0)
            def _():
                out_dma(0, ob1, so1).wait()

            process_chunk(ib1, ob1)
            out_dma(2 * it + 1, ob1, so1).start()
            return carry

        lax.fori_loop(0, niter, iter_body, 0)
        out_dma(0, ob0, so0).wait()
        out_dma(0, ob1, so1).wait()

    return _pool_body


@functools.lru_cache(maxsize=None)
def _pool_call(half):
    mesh = plsc.VectorSubcoreMesh(core_axis_name="c", subcore_axis_name="s")
    return pl.kernel(
        _make_pool_body(half),
        out_type=jax.ShapeDtypeStruct((NG, NH, CG), jnp.float32),
        mesh=mesh,
        compiler_params=pltpu.CompilerParams(
            needs_layout_passes=False, use_tc_tiling_on_sc=False),
        scratch_types=[
            pltpu.VMEM((4001 * PSTRIDE,), jnp.int32),
            pltpu.VMEM((CHUNK, L), jnp.int32),
            pltpu.VMEM((CHUNK, L), jnp.int32),
            pltpu.VMEM((CG * 16,), jnp.float32),
            pltpu.VMEM((CHUNK, CG), jnp.float32),
            pltpu.VMEM((CHUNK, CG), jnp.float32),
            pltpu.SemaphoreType.DMA,
            pltpu.SemaphoreType.DMA,
            pltpu.SemaphoreType.DMA,
            pltpu.SemaphoreType.DMA,
        ],
    )


def _mlp_body(u, f, ct, p, w1, b1, w2, b2, w3, b3, o):
    acc = jnp.dot(u[...], w1[0:DU, :], preferred_element_type=jnp.float32)
    acc += jnp.dot(f[...], w1[DU:DU + DF, :],
                   preferred_element_type=jnp.float32)
    acc += jnp.dot(ct[...], w1[DU + DF:DU + DF + DC, :],
                   preferred_element_type=jnp.float32)
    pb = p[...] * (1.0 / L)
    base = DU + DF + DC
    for gg in range(NG):
        acc += jnp.dot(pb[gg], w1[base + CG * gg:base + CG * (gg + 1), :],
                       preferred_element_type=jnp.float32)
    h = jax.nn.relu(acc + b1[...])
    h2 = jax.nn.relu(jnp.dot(h, w2[...], preferred_element_type=jnp.float32)
                     + b2[...])
    o[...] = jnp.dot(h2, w3[...], preferred_element_type=jnp.float32) + b3[...]


def _mlp(half, user_out, feed_out, city_out, pooled,
         W1, b1, W2, b2, W3, b3):
    BB = 512
    off = half * (NH // BB)
    grid = (NH // BB,)
    return pl.pallas_call(
        _mlp_body,
        grid=grid,
        in_specs=[
            pl.BlockSpec((BB, DU), lambda i: (i + off, 0)),
            pl.BlockSpec((BB, DF), lambda i: (i + off, 0)),
            pl.BlockSpec((BB, DC), lambda i: (i + off, 0)),
            pl.BlockSpec((NG, BB, CG), lambda i: (0, i, 0)),
            pl.BlockSpec((192, 64), lambda i: (0, 0)),
            pl.BlockSpec((1, 64), lambda i: (0, 0)),
            pl.BlockSpec((64, 32), lambda i: (0, 0)),
            pl.BlockSpec((1, 32), lambda i: (0, 0)),
            pl.BlockSpec((32, 2), lambda i: (0, 0)),
            pl.BlockSpec((1, 2), lambda i: (0, 0)),
        ],
        out_specs=pl.BlockSpec((BB, 2), lambda i: (i, 0)),
        out_shape=jax.ShapeDtypeStruct((NH, 2), jnp.float32),
    )(user_out, feed_out, city_out, pooled, W1, b1, W2, b2, W3, b3)


def kernel(user, feed, city, item_emb_seq, user_table, feed_table,
           city_table, W1, b1, W2, b2, W3, b3):
    user = user.astype(jnp.int32)
    feed = feed.astype(jnp.int32)
    city = city.astype(jnp.int32)
    item = item_emb_seq.astype(jnp.int32)
    pk = jax.lax.bitcast_convert_type(
        feed_table.astype(jnp.bfloat16).reshape(4000, 32, 2), jnp.int32)
    pk = jnp.concatenate([pk, jnp.zeros((1, 32), jnp.int32)], axis=0)
    feed_g = pk.reshape(4001, NG, PCG).transpose(1, 0, 2)
    feed_g = jnp.pad(feed_g, ((0, 0), (0, 0), (0, PSTRIDE - PCG)))
    feed_g = feed_g.reshape(NG, 4001 * PSTRIDE)

    b1r, b2r, b3r = b1.reshape(1, -1), b2.reshape(1, -1), b3.reshape(1, -1)
    uo, fo, co = _lookup_call()(user, feed, city, user_table, feed_table,
                                city_table)
    zdep = jax.lax.bitcast_convert_type(uo[0, 0], jnp.int32) & 0
    feed_g = feed_g ^ zdep
    res = []
    for s in range(NSPLIT):
        po = _pool_call(s)(item, feed_g)
        res.append(_mlp(s, uo, fo, co, po, W1, b1r, W2, b2r, W3, b3r))
    return jnp.concatenate(res, axis=0)


# speedup vs baseline: 1.0564x; 1.0564x over previous
import functools

import jax
import jax.numpy as jnp
from jax import lax
from jax.experimental import pallas as pl
from jax.experimental.pallas import tpu as pltpu
from jax.experimental.pallas import tpu_sc as plsc


B = 16384
L = 200
DU = 32
DF = 64
DC = 32
NG = 4
CG = 16
NW = 32
CHUNK = 32
SCHUNK = 128
NJ = L // 16
REM = L - NJ * 16
PCG = 8
PSTRIDE = 9
ZROW = 4000
NSPLIT = 2
NH = B // NSPLIT
SROWS = B // NW
NSC = SROWS // SCHUNK


def _lookup_body(user_r, feed_r, city_r, utab_r, ftab_r, ctab_r,
                 uo_r, fo_r, co_r,
                 uidx, fidx, cidx, ubuf, fbuf, cbuf, sg):
    wid = lax.axis_index("s") * 2 + lax.axis_index("c")
    r0 = wid * SROWS
    pltpu.sync_copy(user_r.at[pl.ds(r0, SROWS)], uidx)
    pltpu.sync_copy(feed_r.at[pl.ds(r0, SROWS)], fidx)
    pltpu.sync_copy(city_r.at[pl.ds(r0, SROWS)], cidx)

    def lookup_dmas():
        for sc in range(NSC):
            lo = sc * SCHUNK
            yield pltpu.make_async_copy(
                utab_r.at[uidx.at[pl.ds(lo, SCHUNK)]],
                ubuf.at[pl.ds(lo, SCHUNK), :], sg)
            yield pltpu.make_async_copy(
                ftab_r.at[fidx.at[pl.ds(lo, SCHUNK)]],
                fbuf.at[pl.ds(lo, SCHUNK), :], sg)
            yield pltpu.make_async_copy(
                ctab_r.at[cidx.at[pl.ds(lo, SCHUNK)]],
                cbuf.at[pl.ds(lo, SCHUNK), :], sg)

    for d in lookup_dmas():
        d.start()
    for d in lookup_dmas():
        d.wait()
    pltpu.sync_copy(ubuf, uo_r.at[pl.ds(r0, SROWS), :])
    pltpu.sync_copy(fbuf, fo_r.at[pl.ds(r0, SROWS), :])
    pltpu.sync_copy(cbuf, co_r.at[pl.ds(r0, SROWS), :])


@functools.lru_cache(maxsize=None)
def _lookup_call():
    mesh = plsc.VectorSubcoreMesh(core_axis_name="c", subcore_axis_name="s")
    return pl.kernel(
        _lookup_body,
        out_type=(
            jax.ShapeDtypeStruct((B, DU), jnp.float32),
            jax.ShapeDtypeStruct((B, DF), jnp.float32),
            jax.ShapeDtypeStruct((B, DC), jnp.float32),
        ),
        mesh=mesh,
        compiler_params=pltpu.CompilerParams(
            needs_layout_passes=False, use_tc_tiling_on_sc=False),
        scratch_types=[
            pltpu.VMEM((SROWS,), jnp.int32),
            pltpu.VMEM((SROWS,), jnp.int32),
            pltpu.VMEM((SROWS,), jnp.int32),
            pltpu.VMEM((SROWS, DU), jnp.float32),
            pltpu.VMEM((SROWS, DF), jnp.float32),
            pltpu.VMEM((SROWS, DC), jnp.float32),
            pltpu.SemaphoreType.DMA,
        ],
    )


def _make_pool_body(half):
    pool_rows = NH // (NW // NG)
    niter = pool_rows // CHUNK // 2

    def _pool_body(item_r, fg_r, po_r,
                   tbl, ib0, ib1, tbuf, ob0, ob1, si0, si1, so0, so1):
        wid = lax.axis_index("s") * 2 + lax.axis_index("c")
        g = lax.rem(wid, NG)
        q = lax.div(wid, NG)

        pltpu.sync_copy(fg_r.at[g], tbl)

        lane = lax.iota(jnp.int32, 16)
        lane_keep = lane >= (16 - REM)
        lane16 = lane * 16

        rows_abs = half * NH + q * pool_rows
        rows_loc = q * pool_rows

        def idx_dma(ch, ib, sem):
            src = item_r.at[pl.ds(rows_abs + ch * CHUNK, CHUNK), :]
            return pltpu.make_async_copy(src, ib, sem)

        def out_dma(ch, ob, sem):
            dst = po_r.at[g, pl.ds(rows_loc + ch * CHUNK, CHUNK), :]
            return pltpu.make_async_copy(ob, dst, sem)

        def process_chunk(ib, ob):
            def row_body(rl, carry):
                accs = [jnp.zeros((32,), jnp.bfloat16) for _ in range(PCG)]
                for j in range(NJ + 1):
                    iv = ib[rl, pl.ds(j * 16 if j < NJ else L - 16, 16)]
                    if j == NJ:
                        iv = jnp.where(lane_keep, iv, ZROW)
                    bidx = iv * PSTRIDE
                    for cc in range(PCG):
                        w = plsc.load_gather(tbl, [bidx + cc])
                        accs[cc] = accs[cc] + plsc.bitcast(w, jnp.bfloat16)
                for cc in range(PCG):
                    a, b = plsc.unpack(
                        accs[cc], format=plsc.PackFormat.INTERLEAVED)
                    tbuf[pl.ds((2 * cc) * 16, 16)] = a
                    tbuf[pl.ds((2 * cc + 1) * 16, 16)] = b
                pv = jnp.zeros((16,), jnp.float32)
                for j in range(16):
                    pv = pv + plsc.load_gather(
                        tbuf, [lane16 + ((lane + j) & 15)])
                ob[rl, :] = pv
                return carry
            lax.fori_loop(0, CHUNK, row_body, 0)

        idx_dma(0, ib0, si0).start()

        def iter_body(it, carry):
            idx_dma(0, ib0, si0).wait()
            idx_dma(2 * it + 1, ib1, si1).start()

            @pl.when(it > 0)
            def _():
                out_dma(0, ob0, so0).wait()

            process_chunk(ib0, ob0)
            out_dma(2 * it, ob0, so0).start()

            idx_dma(0, ib1, si1).wait()

            @pl.when(it < niter - 1)
            def _():
                idx_dma(2 * it + 2, ib0, si0).start()

            @pl.when(it > 0)
            def _():
                out_dma(0, ob1, so1).wait()

            process_chunk(ib1, ob1)
            out_dma(2 * it + 1, ob1, so1).start()
            return carry

        lax.fori_loop(0, niter, iter_body, 0)
        out_dma(0, ob0, so0).wait()
        out_dma(0, ob1, so1).wait()

    return _pool_body


@functools.lru_cache(maxsize=None)
def _pool_call(half):
    mesh = plsc.VectorSubcoreMesh(core_axis_name="c", subcore_axis_name="s")
    return pl.kernel(
        _make_pool_body(half),
        out_type=jax.ShapeDtypeStruct((NG, NH, CG), jnp.float32),
        mesh=mesh,
        compiler_params=pltpu.CompilerParams(
            needs_layout_passes=False, use_tc_tiling_on_sc=False),
        scratch_types=[
            pltpu.VMEM((4001 * PSTRIDE,), jnp.int32),
            pltpu.VMEM((CHUNK, L), jnp.int32),
            pltpu.VMEM((CHUNK, L), jnp.int32),
            pltpu.VMEM((CG * 16,), jnp.float32),
            pltpu.VMEM((CHUNK, CG), jnp.float32),
            pltpu.VMEM((CHUNK, CG), jnp.float32),
            pltpu.SemaphoreType.DMA,
            pltpu.SemaphoreType.DMA,
            pltpu.SemaphoreType.DMA,
            pltpu.SemaphoreType.DMA,
        ],
    )


def _mlp_body(u, f, ct, p, w1, b1, w2, b2, w3, b3, o):
    acc = jnp.dot(u[...], w1[0:DU, :], preferred_element_type=jnp.float32)
    acc += jnp.dot(f[...], w1[DU:DU + DF, :],
                   preferred_element_type=jnp.float32)
    acc += jnp.dot(ct[...], w1[DU + DF:DU + DF + DC, :],
                   preferred_element_type=jnp.float32)
    pb = p[...] * (1.0 / L)
    base = DU + DF + DC
    for gg in range(NG):
        acc += jnp.dot(pb[gg], w1[base + CG * gg:base + CG * (gg + 1), :],
                       preferred_element_type=jnp.float32)
    h = jax.nn.relu(acc + b1[...])
    h2 = jax.nn.relu(jnp.dot(h, w2[...], preferred_element_type=jnp.float32)
                     + b2[...])
    o[...] = jnp.dot(h2, w3[...], preferred_element_type=jnp.float32) + b3[...]


def _mlp(half, user_out, feed_out, city_out, pooled,
         W1, b1, W2, b2, W3, b3):
    BB = 512
    off = half * (NH // BB)
    grid = (NH // BB,)
    return pl.pallas_call(
        _mlp_body,
        grid=grid,
        in_specs=[
            pl.BlockSpec((BB, DU), lambda i: (i + off, 0)),
            pl.BlockSpec((BB, DF), lambda i: (i + off, 0)),
            pl.BlockSpec((BB, DC), lambda i: (i + off, 0)),
            pl.BlockSpec((NG, BB, CG), lambda i: (0, i, 0)),
            pl.BlockSpec((192, 64), lambda i: (0, 0)),
            pl.BlockSpec((1, 64), lambda i: (0, 0)),
            pl.BlockSpec((64, 32), lambda i: (0, 0)),
            pl.BlockSpec((1, 32), lambda i: (0, 0)),
            pl.BlockSpec((32, 2), lambda i: (0, 0)),
            pl.BlockSpec((1, 2), lambda i: (0, 0)),
        ],
        out_specs=pl.BlockSpec((BB, 2), lambda i: (i, 0)),
        out_shape=jax.ShapeDtypeStruct((NH, 2), jnp.float32),
    )(user_out, feed_out, city_out, pooled, W1, b1, W2, b2, W3, b3)


def kernel(user, feed, city, item_emb_seq, user_table, feed_table,
           city_table, W1, b1, W2, b2, W3, b3):
    user = user.astype(jnp.int32)
    feed = feed.astype(jnp.int32)
    city = city.astype(jnp.int32)
    item = item_emb_seq.astype(jnp.int32)
    pk = jax.lax.bitcast_convert_type(
        feed_table.astype(jnp.bfloat16).reshape(4000, 32, 2), jnp.int32)
    pk = jnp.concatenate([pk, jnp.zeros((1, 32), jnp.int32)], axis=0)
    feed_g = pk.reshape(4001, NG, PCG).transpose(1, 0, 2)
    feed_g = jnp.pad(feed_g, ((0, 0), (0, 0), (0, PSTRIDE - PCG)))
    feed_g = feed_g.reshape(NG, 4001 * PSTRIDE)

    b1r, b2r, b3r = b1.reshape(1, -1), b2.reshape(1, -1), b3.reshape(1, -1)
    uo, fo, co = _lookup_call()(user, feed, city, user_table, feed_table,
                                city_table)
    res = []
    for s in range(NSPLIT):
        po = _pool_call(s)(item, feed_g)
        res.append(_mlp(s, uo, fo, co, po, W1, b1r, W2, b2r, W3, b3r))
    return jnp.concatenate(res, axis=0)
